# baseline (speedup 1.0000x reference)
import functools
import jax
import jax.numpy as jnp
from jax import lax
from jax.experimental import pallas as pl
from jax.experimental.pallas import tpu as pltpu
from jax.experimental.pallas import tpu_sc as plsc

D = 1024
B = 4
S = 4096
N = B * S
NW = 32
RPW = N // NW
SPW = S // NW
PC = 4
C = PC * B
G = RPW // C
NIN = 2
NOUT = 4
LANES = 16
DCH = D // LANES
SCALE = 32.0


def _sc_body(x_hbm, tok_hbm, pos_hbm, out_hbm,
             idxall, t0, t1, p0, p1, o0, o1, o2, o3,
             gs0, gs1, ps0, ps1, os0, os1, os2, os3):
    cid = lax.axis_index("c")
    sid = lax.axis_index("s")
    wid = sid * 2 + cid
    ibase = wid * RPW
    s0 = wid * SPW

    pltpu.sync_copy(x_hbm.at[pl.ds(ibase, RPW)], idxall)

    toks = (t0, t1)
    poss = (p0, p1)
    outs = (o0, o1, o2, o3)
    gss = (gs0, gs1)
    pss = (ps0, ps1)
    oss = (os0, os1, os2, os3)

    def issue_in(g, tb):
        pltpu.async_copy(tok_hbm.at[idxall.at[pl.ds(g * C, C)]], toks[tb], gss[tb])
        pltpu.async_copy(pos_hbm.at[pl.ds(s0 + g * PC, PC)], poss[tb], pss[tb])

    def wait_in(g, tb):
        pltpu.make_async_copy(
            tok_hbm.at[idxall.at[pl.ds(g * C, C)]], toks[tb], gss[tb]).wait()
        pltpu.make_async_copy(
            pos_hbm.at[pl.ds(s0 + g * PC, PC)], poss[tb], pss[tb]).wait()

    def issue_wb(g, ob):
        for b in range(B):
            pltpu.async_copy(
                outs[ob].at[pl.ds(b * PC, PC)],
                out_hbm.at[pl.ds(b * S + s0 + g * PC, PC)], oss[ob])

    def wait_wb(g, ob):
        for b in range(B):
            pltpu.make_async_copy(
                outs[ob].at[pl.ds(b * PC, PC)],
                out_hbm.at[pl.ds(b * S + s0 + g * PC, PC)], oss[ob]).wait()

    issue_in(0, 0)
    issue_in(1, 1)

    def quad_body(i, carry):
        for bb in range(NOUT):
            g = i * NOUT + bb
            tb = bb % NIN
            ob = bb
            ob2 = (bb + 2) % NOUT
            if bb < 2:
                @pl.when(i >= 1)
                def _():
                    wait_wb(g - 2, ob2)
            else:
                wait_wb(g - 2, ob2)
            wait_in(g, tb)
            tokb, posb, outb = toks[tb], poss[tb], outs[ob]

            def dcol(d, rc):
                dsl = pl.ds(d * LANES, LANES)
                for sl in range(PC):
                    pv = posb[sl, dsl]
                    for b in range(B):
                        r = b * PC + sl
                        outb[r, dsl] = tokb[r, dsl] * SCALE + pv
                return rc

            lax.fori_loop(0, DCH, dcol, 0)
            issue_wb(g, ob)
            if bb < 2:
                issue_in(g + 2, tb)
            else:
                @pl.when(i < (G // NOUT - 1))
                def _():
                    issue_in(g + 2, tb)
        return carry

    lax.fori_loop(0, G // NOUT, quad_body, 0)
    wait_wb(G - 2, 2)
    wait_wb(G - 1, 3)


@jax.jit
def _run(x_perm, token_table, pos_table):
    mesh = plsc.VectorSubcoreMesh(core_axis_name="c", subcore_axis_name="s")
    k = pl.kernel(
        _sc_body,
        out_type=jax.ShapeDtypeStruct((N, D), jnp.float32),
        mesh=mesh,
        scratch_types=(
            [pltpu.VMEM((RPW,), jnp.int32)]
            + [pltpu.VMEM((C, D), jnp.float32) for _ in range(NIN)]
            + [pltpu.VMEM((PC, D), jnp.float32) for _ in range(NIN)]
            + [pltpu.VMEM((C, D), jnp.float32) for _ in range(NOUT)]
            + [pltpu.SemaphoreType.DMA for _ in range(2 * NIN + NOUT)]
        ),
    )
    return k(x_perm, token_table, pos_table)


def kernel(x, token_table, pos_table):
    x_perm = x.reshape(B, NW, G, PC).transpose(1, 2, 0, 3).reshape(-1)
    out = _run(x_perm, token_table, pos_table)
    return out.reshape(B, S, D)
